# baseline (speedup 1.0000x reference)
import functools

import jax
import jax.numpy as jnp
from jax import lax
from jax.experimental import pallas as pl
from jax.experimental.pallas import tpu as pltpu
from jax.experimental.pallas import tpu_sc as plsc

D_MODEL = 64
LANES = 16
NUM_CORES = 2
NUM_SUBCORES = 16
NUM_WORKERS = NUM_CORES * NUM_SUBCORES
GROUP = 128
GPC = 2
CHUNK = GPC * GROUP
NBUF = 2
SCALE = 8.0


@functools.partial(jax.jit, static_argnums=(2, 3))
def _embed(x_grouped, table, per_worker, n_groups):
    mesh = plsc.VectorSubcoreMesh(core_axis_name="c", subcore_axis_name="s")
    total_rows = NUM_WORKERS * per_worker
    n_chunks = per_worker // CHUNK

    @functools.partial(
        pl.kernel,
        mesh=mesh,
        out_type=jax.ShapeDtypeStruct((total_rows, D_MODEL), jnp.float32),
        scratch_types=[
            pltpu.VMEM((n_groups, GROUP), jnp.int32),
            pltpu.VMEM((CHUNK, D_MODEL), jnp.float32),
            pltpu.VMEM((CHUNK, D_MODEL), jnp.float32),
            pltpu.VMEM((CHUNK, D_MODEL), jnp.float32),
            pltpu.VMEM((CHUNK, D_MODEL), jnp.float32),
            pltpu.SemaphoreType.DMA,
            pltpu.SemaphoreType.DMA,
            pltpu.SemaphoreType.DMA,
            pltpu.SemaphoreType.DMA,
        ],
        compiler_params=pltpu.CompilerParams(use_tc_tiling_on_sc=False),
    )
    def k(x_hbm, table_hbm, out_hbm, idx_v, gb0, gb1, ob0, ob1, gs0, gs1, ss0, ss1):
        gbufs = (gb0, gb1)
        obufs = (ob0, ob1)
        gsems = (gs0, gs1)
        ssems = (ss0, ss1)
        wid = lax.axis_index("s") * NUM_CORES + lax.axis_index("c")
        base = wid * per_worker
        pltpu.sync_copy(x_hbm.at[wid], idx_v)

        def fire_gathers(c, b):
            for j in range(GPC):
                pltpu.async_copy(
                    table_hbm.at[idx_v.at[GPC * c + j]],
                    gbufs[b].at[pl.ds(j * GROUP, GROUP)],
                    gsems[b],
                )

        for b in range(NBUF):
            fire_gathers(b, b)

        def pair_body(p, carry):
            for b in range(NBUF):
                c = NBUF * p + b
                for j in range(GPC):
                    pltpu.make_async_copy(
                        table_hbm.at[idx_v.at[0]],
                        gbufs[b].at[pl.ds(j * GROUP, GROUP)],
                        gsems[b],
                    ).wait()

                @pl.when(c >= NBUF)
                def _():
                    pltpu.make_async_copy(
                        obufs[b], out_hbm.at[pl.ds(base, CHUNK)], ssems[b]
                    ).wait()

                def scale_body(i, cc):
                    for j in range(D_MODEL // LANES):
                        sl = pl.ds(j * LANES, LANES)
                        obufs[b][i, sl] = gbufs[b][i, sl] * SCALE
                    return cc

                lax.fori_loop(0, CHUNK, scale_body, 0, unroll=4)

                @pl.when(c + NBUF < n_chunks)
                def _():
                    fire_gathers(c + NBUF, b)

                pltpu.async_copy(
                    obufs[b],
                    out_hbm.at[pl.ds(base + c * CHUNK, CHUNK)],
                    ssems[b],
                )
            return carry

        lax.fori_loop(0, n_chunks // NBUF, pair_body, 0)
        for b in range(NBUF):
            pltpu.make_async_copy(
                obufs[b], out_hbm.at[pl.ds(base, CHUNK)], ssems[b]
            ).wait()

    return k(x_grouped, table)


def kernel(x, embedding_weight):
    batch, seq = x.shape
    total = batch * seq
    per_worker = total // NUM_WORKERS
    n_groups = per_worker // GROUP
    x_grouped = x.reshape(NUM_WORKERS, n_groups, GROUP).astype(jnp.int32)
    out = _embed(x_grouped, embedding_weight, per_worker, n_groups)
    return out.reshape(batch, seq, D_MODEL)


# speedup vs baseline: 1.2599x; 1.2599x over previous
import functools

import jax
import jax.numpy as jnp
from jax import lax
from jax.experimental import pallas as pl
from jax.experimental.pallas import tpu as pltpu
from jax.experimental.pallas import tpu_sc as plsc

D_MODEL = 64
LANES = 16
NUM_CORES = 2
NUM_SUBCORES = 16
NUM_WORKERS = NUM_CORES * NUM_SUBCORES
GROUP = 128
GPC = 2
CHUNK = GPC * GROUP
NBUF = 4
LEAD = 2
SCALE = 8.0


@functools.partial(jax.jit, static_argnums=(2, 3))
def _embed(x_grouped, table, per_worker, n_groups):
    mesh = plsc.VectorSubcoreMesh(core_axis_name="c", subcore_axis_name="s")
    total_rows = NUM_WORKERS * per_worker
    n_chunks = per_worker // CHUNK

    @functools.partial(
        pl.kernel,
        mesh=mesh,
        out_type=jax.ShapeDtypeStruct((total_rows, D_MODEL), jnp.float32),
        scratch_types=[
            pltpu.VMEM((n_groups, GROUP), jnp.int32),
            tuple(pltpu.VMEM((CHUNK, D_MODEL), jnp.float32) for _ in range(NBUF)),
            tuple(pltpu.SemaphoreType.DMA for _ in range(NBUF)),
            tuple(pltpu.SemaphoreType.DMA for _ in range(NBUF)),
        ],
        compiler_params=pltpu.CompilerParams(use_tc_tiling_on_sc=False),
    )
    def k(x_hbm, table_hbm, out_hbm, idx_v, bufs, gsems, ssems):
        wid = lax.axis_index("s") * NUM_CORES + lax.axis_index("c")
        base = wid * per_worker
        pltpu.sync_copy(x_hbm.at[wid], idx_v)

        def fire_gathers(c, b):
            for j in range(GPC):
                pltpu.async_copy(
                    table_hbm.at[idx_v.at[GPC * c + j]],
                    bufs[b].at[pl.ds(j * GROUP, GROUP)],
                    gsems[b],
                )

        for c0 in range(LEAD):
            fire_gathers(c0, c0)

        def ring_body(p, carry):
            for b in range(NBUF):
                c = NBUF * p + b
                fb = (b + LEAD) % NBUF
                for j in range(GPC):
                    pltpu.make_async_copy(
                        table_hbm.at[idx_v.at[0]],
                        bufs[b].at[pl.ds(j * GROUP, GROUP)],
                        gsems[b],
                    ).wait()

                @pl.when(c >= LEAD)
                def _():
                    pltpu.make_async_copy(
                        bufs[fb], out_hbm.at[pl.ds(base, CHUNK)], ssems[fb]
                    ).wait()

                @pl.when(c + LEAD < n_chunks)
                def _():
                    fire_gathers(c + LEAD, fb)

                @plsc.parallel_loop(0, CHUNK, unroll=8)
                def _(i):
                    for j in range(D_MODEL // LANES):
                        sl = pl.ds(j * LANES, LANES)
                        bufs[b][i, sl] = bufs[b][i, sl] * SCALE

                pltpu.async_copy(
                    bufs[b],
                    out_hbm.at[pl.ds(base + c * CHUNK, CHUNK)],
                    ssems[b],
                )
            return carry

        lax.fori_loop(0, n_chunks // NBUF, ring_body, 0)
        for c in (n_chunks - LEAD, n_chunks - 1):
            pltpu.make_async_copy(
                bufs[c % NBUF], out_hbm.at[pl.ds(base, CHUNK)], ssems[c % NBUF]
            ).wait()

    return k(x_grouped, table)


def kernel(x, embedding_weight):
    batch, seq = x.shape
    total = batch * seq
    per_worker = total // NUM_WORKERS
    n_groups = per_worker // GROUP
    x_grouped = x.reshape(NUM_WORKERS, n_groups, GROUP).astype(jnp.int32)
    out = _embed(x_grouped, embedding_weight, per_worker, n_groups)
    return out.reshape(batch, seq, D_MODEL)
